# baseline (speedup 1.0000x reference)
import jax
import jax.numpy as jnp
from jax import lax
from jax.experimental import pallas as pl
from jax.experimental.pallas import tpu as pltpu
from jax.experimental.pallas import tpu_sc as plsc

_N = 10000
_E = 320000
_D_IN = 128
_D_HID = 128
_D_HALF = 64
_D_OUT = 40
_D_OUT_PAD = 48

_NC = 2
_NS = 16
_NW = _NC * _NS
_C = 128
_NCH = 2560
_E_PAD = _NCH * _C
_N_PAD = 10240
_ROWS_PER_TILE = _N_PAD // _NS
_NBUF = 4


def _sc_conv(d: int, col_split: bool, with_deg: bool, src_spmem: bool,
             lg: int, npass: int, hbm_slots: tuple = ()):
  mesh = plsc.VectorSubcoreMesh(core_axis_name="c", subcore_axis_name="s")
  kc = _NCH // _NS if col_split else _NCH // _NW
  ns = lg
  kcp = kc // npass
  assert kcp % ns == 0 and kcp % 2 == 0 and ns % 2 == 0
  if with_deg:
    out_type = (jax.ShapeDtypeStruct((_NC, _N_PAD, d), jnp.float32),
                jax.ShapeDtypeStruct((_NC, _N_PAD), jnp.float32))
  else:
    out_type = jax.ShapeDtypeStruct((_NC, _N_PAD, d), jnp.float32)
  need_sidx2 = bool(hbm_slots) and col_split
  scratch = [
      pltpu.VMEM((kcp, _C), jnp.int32),
      pltpu.VMEM((kcp, _C), jnp.int32),
      pltpu.VMEM((ns, _C, d), jnp.float32),
      pltpu.VMEM((_C,), jnp.float32),
      pltpu.VMEM_SHARED((_N_PAD, d), jnp.float32),
      pltpu.VMEM_SHARED((_N_PAD,), jnp.float32),
  ] + [pltpu.SemaphoreType.DMA] * ns
  if src_spmem:
    scratch.append(pltpu.VMEM_SHARED((_N_PAD, d), jnp.float32))
  if need_sidx2:
    scratch.append(pltpu.VMEM((kcp, _C), jnp.int32))

  def body(h_hbm, *rest):
    if need_sidx2:
      sidx_hbm, sidx2_hbm = rest[0], rest[1]
      rest = rest[2:]
    else:
      sidx_hbm = rest[0]
      rest = rest[1:]
    didx_hbm, zrows_hbm, zdeg_hbm = rest[0], rest[1], rest[2]
    rest = rest[3:]
    if with_deg:
      part_hbm, degp_hbm = rest[0], rest[1]
      scr = rest[2:]
    else:
      part_hbm = rest[0]
      scr = rest[1:]
    sidx_v, didx_v, rows_v, ones_v, acc_sh, dacc_sh = scr[:6]
    gsem = scr[6:6 + ns]
    scr = scr[6 + ns:]
    c = lax.axis_index("c")
    s = lax.axis_index("s")
    row0 = s * _ROWS_PER_TILE

    pltpu.sync_copy(zrows_hbm.at[pl.ds(row0, _ROWS_PER_TILE)],
                    acc_sh.at[pl.ds(row0, _ROWS_PER_TILE)])
    if src_spmem:
      h_src = scr[0]
      scr = scr[1:]
      if col_split and need_sidx2:
        pltpu.sync_copy(h_hbm.at[pl.ds(c * _N_PAD + row0, _ROWS_PER_TILE)],
                        h_src.at[pl.ds(row0, _ROWS_PER_TILE)])
      elif col_split:
        pltpu.sync_copy(h_hbm.at[c, pl.ds(row0, _ROWS_PER_TILE)],
                        h_src.at[pl.ds(row0, _ROWS_PER_TILE)])
      else:
        pltpu.sync_copy(h_hbm.at[pl.ds(row0, _ROWS_PER_TILE)],
                        h_src.at[pl.ds(row0, _ROWS_PER_TILE)])
    else:
      h_src = h_hbm.at[c] if col_split else h_hbm
    sidx2_v = scr[0] if need_sidx2 else None
    if with_deg:
      pltpu.sync_copy(zdeg_hbm.at[pl.ds(row0, _ROWS_PER_TILE)],
                      dacc_sh.at[pl.ds(row0, _ROWS_PER_TILE)])
      for i in range(_C // 16):
        ones_v[pl.ds(16 * i, 16)] = jnp.full((16,), 1.0, jnp.float32)
    chunk0 = s * kc if col_split else (c * _NS + s) * kc
    plsc.subcore_barrier()

    def gsrc(b, j):
      if b in hbm_slots:
        if need_sidx2:
          return h_hbm.at[sidx2_v.at[j]]
        return h_hbm.at[sidx_v.at[j]]
      return h_src.at[sidx_v.at[j]]

    def pstep(g, b, last):
      j = g * ns + b
      pltpu.make_async_copy(gsrc(b, j), rows_v.at[b],
                            gsem[b]).wait()
      pltpu.sync_copy(rows_v.at[b], acc_sh.at[didx_v.at[j]], add=True)
      if with_deg:
        if col_split:
          if b % 2 == 0:
            dj = c * (kcp // 2) + g * (ns // 2) + b // 2
            pltpu.sync_copy(ones_v, dacc_sh.at[didx_v.at[dj]], add=True)
        else:
          pltpu.sync_copy(ones_v, dacc_sh.at[didx_v.at[j]], add=True)
      if not last:
        pltpu.async_copy(gsrc(b, j + lg), rows_v.at[b], gsem[b])

    for p in range(npass):
      pbase = chunk0 + p * kcp
      pltpu.sync_copy(sidx_hbm.at[pl.ds(pbase, kcp)], sidx_v)
      if need_sidx2:
        pltpu.sync_copy(sidx2_hbm.at[c, pl.ds(pbase, kcp)], sidx2_v)
      pltpu.sync_copy(didx_hbm.at[pl.ds(pbase, kcp)], didx_v)

      for b in range(lg):
        pltpu.async_copy(gsrc(b, b), rows_v.at[b], gsem[b])

      def group(g, carry):
        for b in range(ns):
          pstep(g, b, False)
        return carry

      lax.fori_loop(0, kcp // ns - 1, group, 0)
      for b in range(ns):
        pstep(kcp // ns - 1, b, True)
    plsc.subcore_barrier()

    pltpu.sync_copy(acc_sh.at[pl.ds(row0, _ROWS_PER_TILE)],
                    part_hbm.at[c, pl.ds(row0, _ROWS_PER_TILE)])
    if with_deg:
      pltpu.sync_copy(dacc_sh.at[pl.ds(row0, _ROWS_PER_TILE)],
                      degp_hbm.at[c, pl.ds(row0, _ROWS_PER_TILE)])

  params = pltpu.CompilerParams(use_tc_tiling_on_sc=False)
  return pl.kernel(body, out_type=out_type, mesh=mesh, scratch_types=scratch,
                   compiler_params=params, name=f"sc_conv_d{d}")


def _erf(z):
  a1, a2, a3, a4, a5 = (0.254829592, -0.284496736, 1.421413741,
                        -1.453152027, 1.061405429)
  p = 0.3275911
  az = jnp.abs(z)
  t = 1.0 / (1.0 + p * az)
  poly = t * (a1 + t * (a2 + t * (a3 + t * (a4 + t * a5))))
  e = 1.0 - poly * jnp.exp(-az * az)
  return jnp.sign(z) * e


def _mm_body(x_ref, w_ref, o_ref):
  o_ref[0] = jnp.dot(x_ref[...], w_ref[0],
                     preferred_element_type=jnp.float32)


def _mid_body(part_ref, deg_ref, b0_ref, w1_ref, o_ref):
  deg = jnp.maximum(deg_ref[0] + deg_ref[1], 1.0)
  agg = jnp.concatenate([part_ref[0], part_ref[1]], axis=-1)
  h = agg / deg + b0_ref[...]
  m = jnp.mean(h, axis=-1, keepdims=True)
  hc = h - m
  v = jnp.mean(hc * hc, axis=-1, keepdims=True)
  hn = hc / jnp.sqrt(v + 1e-5)
  g = 0.5 * hn * (1.0 + _erf(hn * 0.7071067811865476))
  o_ref[...] = jnp.dot(g, w1_ref[...], preferred_element_type=jnp.float32)


def _out_body(part_ref, deg_ref, b1_ref, o_ref):
  deg = jnp.maximum(deg_ref[0] + deg_ref[1], 1.0)
  t = (part_ref[0] + part_ref[1]) / deg + b1_ref[...]
  col = lax.broadcasted_iota(jnp.int32, t.shape, 1)
  t = jnp.where(col < _D_OUT, t, -1e30)
  mx = jnp.max(t, axis=-1, keepdims=True)
  lse = mx + jnp.log(jnp.sum(jnp.exp(t - mx), axis=-1, keepdims=True))
  o_ref[...] = t - lse


_RB = 512
_G = _N_PAD // _RB


def _tc_matmul(x, w):
  return pl.pallas_call(
      _mm_body,
      grid=(_NC, _G),
      in_specs=[pl.BlockSpec((_RB, _D_IN), lambda h, i: (i, 0)),
                pl.BlockSpec((1, _D_IN, _D_HALF), lambda h, i: (h, 0, 0))],
      out_specs=pl.BlockSpec((1, _RB, _D_HALF), lambda h, i: (h, i, 0)),
      out_shape=jax.ShapeDtypeStruct((_NC, _N_PAD, _D_HALF), jnp.float32),
  )(x, w)


def _tc_mid(part, deg, b0, w1p):
  return pl.pallas_call(
      _mid_body,
      grid=(_G,),
      in_specs=[pl.BlockSpec((_NC, _RB, _D_HALF), lambda i: (0, i, 0)),
                pl.BlockSpec((_NC, _RB, 1), lambda i: (0, i, 0)),
                pl.BlockSpec((1, _D_HID), lambda i: (0, 0)),
                pl.BlockSpec((_D_HID, _D_OUT_PAD), lambda i: (0, 0))],
      out_specs=pl.BlockSpec((_RB, _D_OUT_PAD), lambda i: (i, 0)),
      out_shape=jax.ShapeDtypeStruct((_N_PAD, _D_OUT_PAD), jnp.float32),
  )(part, deg, b0, w1p)


def _tc_out(part, deg, b1p):
  return pl.pallas_call(
      _out_body,
      grid=(_G,),
      in_specs=[pl.BlockSpec((_NC, _RB, _D_OUT_PAD), lambda i: (0, i, 0)),
                pl.BlockSpec((_NC, _RB, 1), lambda i: (0, i, 0)),
                pl.BlockSpec((1, _D_OUT_PAD), lambda i: (0, 0))],
      out_specs=pl.BlockSpec((_RB, _D_OUT_PAD), lambda i: (i, 0)),
      out_shape=jax.ShapeDtypeStruct((_N_PAD, _D_OUT_PAD), jnp.float32),
  )(part, deg, b1p)


@jax.jit
def kernel(x, edge_index, W0, b0, W1, b1):
  src = edge_index[0]
  dst = edge_index[1]
  pad = _E_PAD - _E
  src_p = jnp.concatenate([src, jnp.zeros((pad,), jnp.int32)])
  dst_p = jnp.concatenate([dst, jnp.full((pad,), _N, jnp.int32)])
  sidx = src_p.reshape(_NCH, _C)
  didx = dst_p.reshape(_NCH, _C)

  x_pad = jnp.zeros((_N_PAD, _D_IN), jnp.float32).at[:_N].set(x)
  w1p = jnp.zeros((_D_HID, _D_OUT_PAD), jnp.float32).at[:, :_D_OUT].set(W1)
  b1p = jnp.zeros((1, _D_OUT_PAD), jnp.float32).at[0, :_D_OUT].set(b1)
  b0r = b0.reshape(1, _D_HID)

  zrows = jnp.zeros((_N_PAD, _D_HALF), jnp.float32)
  zrows2 = jnp.zeros((_N_PAD, _D_OUT_PAD), jnp.float32)
  zdeg = jnp.zeros((_N_PAD,), jnp.float32)

  w0h = jnp.stack([W0[:, :_D_HALF], W0[:, _D_HALF:]])
  h1 = _tc_matmul(x_pad, w0h)
  h1f = h1.reshape(_NC * _N_PAD, _D_HALF)
  sidx2 = jnp.stack([sidx, sidx + _N_PAD])
  part1, degp = _sc_conv(_D_HALF, True, True, True, 4, 4, (1, 3))(
      h1f, sidx, sidx2, didx, zrows, zdeg)
  deg = degp.reshape(_NC, _N_PAD, 1)
  h2 = _tc_mid(part1, deg, b0r, w1p)
  part2 = _sc_conv(_D_OUT_PAD, False, False, True, 8, 2, (1, 3, 5))(
      h2, sidx, didx, zrows2, zdeg)
  out = _tc_out(part2, deg, b1p)
  return out[:_N, :_D_OUT]


# speedup vs baseline: 1.1948x; 1.1948x over previous
import jax
import jax.numpy as jnp
from jax import lax
from jax.experimental import pallas as pl
from jax.experimental.pallas import tpu as pltpu
from jax.experimental.pallas import tpu_sc as plsc

_N = 10000
_E = 320000
_D_IN = 128
_D_HID = 128
_D_HALF = 64
_D_OUT = 40
_D_OUT_PAD = 48

_NC = 2
_NS = 16
_NW = _NC * _NS
_C = 128
_NCH = 2560
_E_PAD = _NCH * _C
_N_PAD = 10240
_ROWS_PER_TILE = _N_PAD // _NS
_NBUF = 4


def _sc_conv(d: int, col_split: bool, with_deg: bool, src_spmem: bool,
             lg: int, npass: int, ls: int = 0):
  mesh = plsc.VectorSubcoreMesh(core_axis_name="c", subcore_axis_name="s")
  kc = _NCH // _NS if col_split else _NCH // _NW
  ns = lg + ls
  kcp = kc // npass
  assert kcp % ns == 0 and kcp % 2 == 0 and ns % 2 == 0
  if with_deg:
    out_type = (jax.ShapeDtypeStruct((_NC, _N_PAD, d), jnp.float32),
                jax.ShapeDtypeStruct((_NC, _N_PAD), jnp.float32))
  else:
    out_type = jax.ShapeDtypeStruct((_NC, _N_PAD, d), jnp.float32)
  scratch = [
      pltpu.VMEM((kcp, _C), jnp.int32),
      pltpu.VMEM((kcp, _C), jnp.int32),
      pltpu.VMEM((ns, _C, d), jnp.float32),
      pltpu.VMEM((_C,), jnp.float32),
      pltpu.VMEM_SHARED((_N_PAD, d), jnp.float32),
      pltpu.VMEM_SHARED((_N_PAD,), jnp.float32),
  ] + [pltpu.SemaphoreType.DMA] * (ns + (ns if ls else 0))
  if src_spmem:
    scratch.append(pltpu.VMEM_SHARED((_N_PAD, d), jnp.float32))

  def body(h_hbm, sidx_hbm, didx_hbm, zrows_hbm, zdeg_hbm, *rest):
    if with_deg:
      part_hbm, degp_hbm = rest[0], rest[1]
      scr = rest[2:]
    else:
      part_hbm = rest[0]
      scr = rest[1:]
    sidx_v, didx_v, rows_v, ones_v, acc_sh, dacc_sh = scr[:6]
    gsem = scr[6:6 + ns]
    ssem = scr[6 + ns:6 + 2 * ns] if ls else None
    if ls:
      scr = (scr[:6 + ns]) + scr[6 + 2 * ns:]
    c = lax.axis_index("c")
    s = lax.axis_index("s")
    row0 = s * _ROWS_PER_TILE

    pltpu.sync_copy(zrows_hbm.at[pl.ds(row0, _ROWS_PER_TILE)],
                    acc_sh.at[pl.ds(row0, _ROWS_PER_TILE)])
    if src_spmem:
      h_src = scr[6 + ns]
      if col_split:
        pltpu.sync_copy(h_hbm.at[c, pl.ds(row0, _ROWS_PER_TILE)],
                        h_src.at[pl.ds(row0, _ROWS_PER_TILE)])
      else:
        pltpu.sync_copy(h_hbm.at[pl.ds(row0, _ROWS_PER_TILE)],
                        h_src.at[pl.ds(row0, _ROWS_PER_TILE)])
    else:
      h_src = h_hbm.at[c] if col_split else h_hbm
    if with_deg:
      pltpu.sync_copy(zdeg_hbm.at[pl.ds(row0, _ROWS_PER_TILE)],
                      dacc_sh.at[pl.ds(row0, _ROWS_PER_TILE)])
      for i in range(_C // 16):
        ones_v[pl.ds(16 * i, 16)] = jnp.full((16,), 1.0, jnp.float32)
    chunk0 = s * kc if col_split else (c * _NS + s) * kc
    plsc.subcore_barrier()

    def pstep(g, b, first, last):
      j = g * ns + b
      pltpu.make_async_copy(h_src.at[sidx_v.at[j]], rows_v.at[b],
                            gsem[b]).wait()
      if ls:
        pltpu.async_copy(rows_v.at[b], acc_sh.at[didx_v.at[j]], ssem[b],
                         add=True)
      else:
        pltpu.sync_copy(rows_v.at[b], acc_sh.at[didx_v.at[j]], add=True)
      if with_deg:
        if col_split:
          if b % 2 == 0:
            dj = c * (kcp // 2) + g * (ns // 2) + b // 2
            pltpu.sync_copy(ones_v, dacc_sh.at[didx_v.at[dj]], add=True)
        else:
          pltpu.sync_copy(ones_v, dacc_sh.at[didx_v.at[j]], add=True)
      bf = (b + lg) % ns
      if ls and not (first and b < ls):
        m = j - ls
        pltpu.make_async_copy(rows_v.at[bf], acc_sh.at[didx_v.at[m]],
                              ssem[bf]).wait()
      if (not last) or (ls and b < ls):
        pltpu.async_copy(h_src.at[sidx_v.at[j + lg]], rows_v.at[bf], gsem[bf])

    for p in range(npass):
      pbase = chunk0 + p * kcp
      pltpu.sync_copy(sidx_hbm.at[pl.ds(pbase, kcp)], sidx_v)
      pltpu.sync_copy(didx_hbm.at[pl.ds(pbase, kcp)], didx_v)

      for b in range(lg):
        pltpu.async_copy(h_src.at[sidx_v.at[b]], rows_v.at[b], gsem[b])

      def group(g, carry):
        for b in range(ns):
          pstep(g, b, False, False)
        return carry

      for b in range(ns):
        pstep(0, b, True, False)
      lax.fori_loop(1, kcp // ns - 1, group, 0)
      for b in range(ns):
        pstep(kcp // ns - 1, b, False, True)
      for i in range(ls):
        m = kcp - ls + i
        pltpu.make_async_copy(rows_v.at[m % ns], acc_sh.at[didx_v.at[m]],
                              ssem[m % ns]).wait()
    plsc.subcore_barrier()

    pltpu.sync_copy(acc_sh.at[pl.ds(row0, _ROWS_PER_TILE)],
                    part_hbm.at[c, pl.ds(row0, _ROWS_PER_TILE)])
    if with_deg:
      pltpu.sync_copy(dacc_sh.at[pl.ds(row0, _ROWS_PER_TILE)],
                      degp_hbm.at[c, pl.ds(row0, _ROWS_PER_TILE)])

  params = pltpu.CompilerParams(use_tc_tiling_on_sc=False)
  return pl.kernel(body, out_type=out_type, mesh=mesh, scratch_types=scratch,
                   compiler_params=params, name=f"sc_conv_d{d}")


def _erf(z):
  a1, a2, a3, a4, a5 = (0.254829592, -0.284496736, 1.421413741,
                        -1.453152027, 1.061405429)
  p = 0.3275911
  az = jnp.abs(z)
  t = 1.0 / (1.0 + p * az)
  poly = t * (a1 + t * (a2 + t * (a3 + t * (a4 + t * a5))))
  e = 1.0 - poly * jnp.exp(-az * az)
  return jnp.sign(z) * e


def _mm_body(x_ref, w_ref, o_ref):
  o_ref[0] = jnp.dot(x_ref[...], w_ref[0],
                     preferred_element_type=jnp.float32)


def _mid_body(part_ref, deg_ref, b0_ref, w1_ref, o_ref):
  deg = jnp.maximum(deg_ref[0] + deg_ref[1], 1.0)
  agg = jnp.concatenate([part_ref[0], part_ref[1]], axis=-1)
  h = agg / deg + b0_ref[...]
  m = jnp.mean(h, axis=-1, keepdims=True)
  hc = h - m
  v = jnp.mean(hc * hc, axis=-1, keepdims=True)
  hn = hc / jnp.sqrt(v + 1e-5)
  g = 0.5 * hn * (1.0 + _erf(hn * 0.7071067811865476))
  o_ref[...] = jnp.dot(g, w1_ref[...], preferred_element_type=jnp.float32)


def _out_body(part_ref, deg_ref, b1_ref, o_ref):
  deg = jnp.maximum(deg_ref[0] + deg_ref[1], 1.0)
  t = (part_ref[0] + part_ref[1]) / deg + b1_ref[...]
  col = lax.broadcasted_iota(jnp.int32, t.shape, 1)
  t = jnp.where(col < _D_OUT, t, -1e30)
  mx = jnp.max(t, axis=-1, keepdims=True)
  lse = mx + jnp.log(jnp.sum(jnp.exp(t - mx), axis=-1, keepdims=True))
  o_ref[...] = t - lse


_RB = 512
_G = _N_PAD // _RB


def _tc_matmul(x, w):
  return pl.pallas_call(
      _mm_body,
      grid=(_NC, _G),
      in_specs=[pl.BlockSpec((_RB, _D_IN), lambda h, i: (i, 0)),
                pl.BlockSpec((1, _D_IN, _D_HALF), lambda h, i: (h, 0, 0))],
      out_specs=pl.BlockSpec((1, _RB, _D_HALF), lambda h, i: (h, i, 0)),
      out_shape=jax.ShapeDtypeStruct((_NC, _N_PAD, _D_HALF), jnp.float32),
  )(x, w)


def _tc_mid(part, deg, b0, w1p):
  return pl.pallas_call(
      _mid_body,
      grid=(_G,),
      in_specs=[pl.BlockSpec((_NC, _RB, _D_HALF), lambda i: (0, i, 0)),
                pl.BlockSpec((_NC, _RB, 1), lambda i: (0, i, 0)),
                pl.BlockSpec((1, _D_HID), lambda i: (0, 0)),
                pl.BlockSpec((_D_HID, _D_OUT_PAD), lambda i: (0, 0))],
      out_specs=pl.BlockSpec((_RB, _D_OUT_PAD), lambda i: (i, 0)),
      out_shape=jax.ShapeDtypeStruct((_N_PAD, _D_OUT_PAD), jnp.float32),
  )(part, deg, b0, w1p)


def _tc_out(part, deg, b1p):
  return pl.pallas_call(
      _out_body,
      grid=(_G,),
      in_specs=[pl.BlockSpec((_NC, _RB, _D_OUT_PAD), lambda i: (0, i, 0)),
                pl.BlockSpec((_NC, _RB, 1), lambda i: (0, i, 0)),
                pl.BlockSpec((1, _D_OUT_PAD), lambda i: (0, 0))],
      out_specs=pl.BlockSpec((_RB, _D_OUT_PAD), lambda i: (i, 0)),
      out_shape=jax.ShapeDtypeStruct((_N_PAD, _D_OUT_PAD), jnp.float32),
  )(part, deg, b1p)


@jax.jit
def kernel(x, edge_index, W0, b0, W1, b1):
  src = edge_index[0]
  dst = edge_index[1]
  pad = _E_PAD - _E
  src_p = jnp.concatenate([src, jnp.zeros((pad,), jnp.int32)])
  dst_p = jnp.concatenate([dst, jnp.full((pad,), _N, jnp.int32)])
  sidx = src_p.reshape(_NCH, _C)
  didx = dst_p.reshape(_NCH, _C)

  x_pad = jnp.zeros((_N_PAD, _D_IN), jnp.float32).at[:_N].set(x)
  w1p = jnp.zeros((_D_HID, _D_OUT_PAD), jnp.float32).at[:, :_D_OUT].set(W1)
  b1p = jnp.zeros((1, _D_OUT_PAD), jnp.float32).at[0, :_D_OUT].set(b1)
  b0r = b0.reshape(1, _D_HID)

  zrows = jnp.zeros((_N_PAD, _D_HALF), jnp.float32)
  zrows2 = jnp.zeros((_N_PAD, _D_OUT_PAD), jnp.float32)
  zdeg = jnp.zeros((_N_PAD,), jnp.float32)

  w0h = jnp.stack([W0[:, :_D_HALF], W0[:, _D_HALF:]])
  h1 = _tc_matmul(x_pad, w0h)
  part1, degp = _sc_conv(_D_HALF, True, True, True, 2, 4, 2)(
      h1, sidx, didx, zrows, zdeg)
  deg = degp.reshape(_NC, _N_PAD, 1)
  h2 = _tc_mid(part1, deg, b0r, w1p)
  part2 = _sc_conv(_D_OUT_PAD, False, False, True, 4, 2, 4)(
      h2, sidx, didx, zrows2, zdeg)
  out = _tc_out(part2, deg, b1p)
  return out[:_N, :_D_OUT]
